# baseline (speedup 1.0000x reference)
import jax
import jax.numpy as jnp
from jax import lax
from jax.experimental import pallas as pl
from jax.experimental.pallas import tpu as pltpu
from jax.experimental.pallas import tpu_sc as plsc

VOCAB = 100000
EMBED = 128
B = 16384
L = 20

NW = 32
ROWS_PER_W = B // NW
CHUNK = 16
CL = CHUNK * L
NCHUNK = ROWS_PER_W // CHUNK
PIECES = (128, 128, 64)
D16 = EMBED // 16
BLK = 4
NBLK = CHUNK // BLK


def _sc_kernel(center1d, ctx2d, in_tab, out_tab, out,
               center_v, cs0, cs1, cf0, cf1, vc0, vc1, vo0, vo1,
               ob0, ob1, ot0, ot1, ssem0, ssem1, gsem0, gsem1,
               wsem0, wsem1):
    wid = lax.axis_index("s") * 2 + lax.axis_index("c")
    lanes = lax.iota(jnp.int32, 16)
    cs_bufs, cf_bufs = (cs0, cs1), (cf0, cf1)
    vc_bufs, vo_bufs = (vc0, vc1), (vo0, vo1)
    ob_bufs, ot_bufs = (ob0, ob1), (ot0, ot1)
    ssems, gsems, wsems = (ssem0, ssem1), (gsem0, gsem1), (wsem0, wsem1)
    row_base = wid * ROWS_PER_W

    pltpu.sync_copy(center1d.at[pl.ds(row_base, ROWS_PER_W)], center_v)

    def stage_copy(ci, b):
        return pltpu.make_async_copy(
            ctx2d.at[pl.ds(row_base + ci * CHUNK, CHUNK)],
            cs_bufs[b], ssems[b])

    def flatten(b):
        cs, cf = cs_bufs[b], cf_bufs[b]
        for r in range(CHUNK):
            cf[pl.ds(r * L, 16)] = cs[r, pl.ds(0, 16)]
            cf[pl.ds(r * L + L - 16, 16)] = cs[r, pl.ds(L - 16, 16)]

    def gather_copies(ci, b):
        cps = [pltpu.make_async_copy(
            in_tab.at[center_v.at[pl.ds(ci * CHUNK, CHUNK)]],
            vc_bufs[b], gsems[b])]
        off = 0
        for n in PIECES:
            cps.append(pltpu.make_async_copy(
                out_tab.at[cf_bufs[b].at[pl.ds(off, n)]],
                vo_bufs[b].at[pl.ds(off, n)], gsems[b]))
            off += n
        return cps

    def issue_gathers(ci, b):
        for cp in gather_copies(ci, b):
            cp.start()

    def wait_gathers(ci, b):
        for cp in gather_copies(ci, b):
            cp.wait()

    def wb_copy(ci, b):
        return pltpu.make_async_copy(
            ot_bufs[b], out.at[pl.ds(row_base + ci * CHUNK, CHUNK)],
            wsems[b])

    stage_copy(0, 0).start()
    stage_copy(0, 0).wait()
    flatten(0)
    issue_gathers(0, 0)
    stage_copy(1, 1).start()

    def pair_body(c2, _):
        for b in (0, 1):
            c = c2 * 2 + b
            vc_buf, vo_buf = vc_bufs[b], vo_bufs[b]
            ob, ot = ob_bufs[b], ot_bufs[b]

            @pl.when(c + 1 < NCHUNK)
            def _():
                stage_copy(c + 1, 1 - b).wait()
                flatten(1 - b)
                issue_gathers(c + 1, 1 - b)

            @pl.when(c + 2 < NCHUNK)
            def _():
                stage_copy(c + 2, b).start()

            wait_gathers(c, b)

            @pl.when(c >= 2)
            def _():
                wb_copy(c - 2, b).wait()

            @plsc.parallel_loop(0, NBLK, unroll=2)
            def block_body(bb):
                sums = []
                for ii in range(BLK):
                    i = bb * BLK + ii
                    vc = [vc_buf[i, pl.ds(d * 16, 16)] for d in range(D16)]
                    for l in range(L):
                        r = i * L + l
                        acc = vc[0] * vo_buf[r, pl.ds(0, 16)]
                        for d in range(1, D16):
                            acc += vc[d] * vo_buf[r, pl.ds(d * 16, 16)]
                        sums.append(jnp.sum(acc))
                for v in range(BLK * L // 16):
                    vec = jnp.full((16,), sums[v * 16], jnp.float32)
                    for k in range(1, 16):
                        vec = jnp.where(lanes == k, sums[v * 16 + k], vec)
                    ob[pl.ds(bb * (BLK * L) + v * 16, 16)] = vec

            @plsc.parallel_loop(0, CHUNK)
            def retile(r):
                ot[r, pl.ds(0, 16)] = ob[pl.ds(r * L, 16)]
                ot[r, pl.ds(L - 16, 16)] = ob[pl.ds(r * L + L - 16, 16)]

            wb_copy(c, b).start()
        return 0

    lax.fori_loop(0, NCHUNK // 2, pair_body, 0)
    wb_copy(NCHUNK - 2, 0).wait()
    wb_copy(NCHUNK - 1, 1).wait()


def kernel(center, context, in_embed, out_embed):
    center1d = center.astype(jnp.int32)
    ctx2d = context.astype(jnp.int32)

    mesh = plsc.VectorSubcoreMesh(core_axis_name="c", subcore_axis_name="s")
    f = pl.kernel(
        _sc_kernel,
        out_type=jax.ShapeDtypeStruct((B, L), jnp.float32),
        mesh=mesh,
        compiler_params=pltpu.CompilerParams(needs_layout_passes=False),
        scratch_types=[
            pltpu.VMEM((ROWS_PER_W,), jnp.int32),
            pltpu.VMEM((CHUNK, L), jnp.int32),
            pltpu.VMEM((CHUNK, L), jnp.int32),
            pltpu.VMEM((CL,), jnp.int32),
            pltpu.VMEM((CL,), jnp.int32),
            pltpu.VMEM((CHUNK, EMBED), jnp.float32),
            pltpu.VMEM((CHUNK, EMBED), jnp.float32),
            pltpu.VMEM((CL, EMBED), jnp.float32),
            pltpu.VMEM((CL, EMBED), jnp.float32),
            pltpu.VMEM((CL,), jnp.float32),
            pltpu.VMEM((CL,), jnp.float32),
            pltpu.VMEM((CHUNK, L), jnp.float32),
            pltpu.VMEM((CHUNK, L), jnp.float32),
            pltpu.SemaphoreType.DMA,
            pltpu.SemaphoreType.DMA,
            pltpu.SemaphoreType.DMA,
            pltpu.SemaphoreType.DMA,
            pltpu.SemaphoreType.DMA,
            pltpu.SemaphoreType.DMA,
        ],
    )
    return f(center1d, ctx2d, in_embed, out_embed)


# speedup vs baseline: 1.9931x; 1.9931x over previous
import jax
import jax.numpy as jnp
from jax import lax
from jax.experimental import pallas as pl
from jax.experimental.pallas import tpu as pltpu
from jax.experimental.pallas import tpu_sc as plsc

VOCAB = 100000
EMBED = 128
B = 16384
L = 20

NW = 32
ROWS_PER_W = B // NW
CHUNK = 16
CL = CHUNK * L
NCHUNK = ROWS_PER_W // CHUNK
PIECES = (128, 128, 64)
D16 = EMBED // 16
BLK = 4
NBLK = CHUNK // BLK


def _sc_kernel(center1d, ctx2d, in_tab, out_tab, out,
               center_v, cs0, cs1, cf0, cf1, vc0, vc1, vo0, vo1,
               ob0, ob1, ot0, ot1, ssem0, ssem1, gsem0, gsem1,
               wsem0, wsem1):
    wid = lax.axis_index("s") * 2 + lax.axis_index("c")
    lanes = lax.iota(jnp.int32, 16)
    cs_bufs, cf_bufs = (cs0, cs1), (cf0, cf1)
    vc_bufs, vo_bufs = (vc0, vc1), (vo0, vo1)
    ob_bufs, ot_bufs = (ob0, ob1), (ot0, ot1)
    ssems, gsems, wsems = (ssem0, ssem1), (gsem0, gsem1), (wsem0, wsem1)
    row_base = wid * ROWS_PER_W

    pltpu.sync_copy(center1d.at[pl.ds(row_base, ROWS_PER_W)], center_v)

    def stage_copy(ci, b):
        return pltpu.make_async_copy(
            ctx2d.at[pl.ds(row_base + ci * CHUNK, CHUNK)],
            cs_bufs[b], ssems[b])

    def flatten(b):
        cs, cf = cs_bufs[b], cf_bufs[b]
        for r in range(CHUNK):
            cf[pl.ds(r * L, 16)] = cs[r, pl.ds(0, 16)]
            cf[pl.ds(r * L + L - 16, 16)] = cs[r, pl.ds(L - 16, 16)]

    def gather_copies(ci, b):
        cps = [pltpu.make_async_copy(
            in_tab.at[center_v.at[pl.ds(ci * CHUNK, CHUNK)]],
            vc_bufs[b], gsems[b])]
        off = 0
        for n in PIECES:
            cps.append(pltpu.make_async_copy(
                out_tab.at[cf_bufs[b].at[pl.ds(off, n)]],
                vo_bufs[b].at[pl.ds(off, n)], gsems[b]))
            off += n
        return cps

    def issue_gathers(ci, b):
        for cp in gather_copies(ci, b):
            cp.start()

    def wait_gathers(ci, b):
        for cp in gather_copies(ci, b):
            cp.wait()

    def wb_copy(ci, b):
        return pltpu.make_async_copy(
            ot_bufs[b], out.at[pl.ds(row_base + ci * CHUNK, CHUNK)],
            wsems[b])

    stage_copy(0, 0).start()
    stage_copy(0, 0).wait()
    flatten(0)
    issue_gathers(0, 0)
    stage_copy(1, 1).start()

    def pair_body(c2, _):
        for b in (0, 1):
            c = c2 * 2 + b
            vc_buf, vo_buf = vc_bufs[b], vo_bufs[b]
            ob, ot = ob_bufs[b], ot_bufs[b]

            @pl.when(c + 1 < NCHUNK)
            def _():
                stage_copy(c + 1, 1 - b).wait()
                flatten(1 - b)
                issue_gathers(c + 1, 1 - b)

            @pl.when(c + 2 < NCHUNK)
            def _():
                stage_copy(c + 2, b).start()

            wait_gathers(c, b)

            @pl.when(c >= 2)
            def _():
                wb_copy(c - 2, b).wait()

            @plsc.parallel_loop(0, NBLK)
            def block_body(bb):
                sums = []
                for ii in range(BLK):
                    i = bb * BLK + ii
                    vc = [vc_buf[i, pl.ds(d * 16, 16)] for d in range(D16)]
                    for l in range(L):
                        r = i * L + l
                        acc0 = vc[0] * vo_buf[r, pl.ds(0, 16)]
                        acc1 = vc[1] * vo_buf[r, pl.ds(16, 16)]
                        for d in range(2, D16, 2):
                            acc0 += vc[d] * vo_buf[r, pl.ds(d * 16, 16)]
                            acc1 += vc[d + 1] * vo_buf[r, pl.ds((d + 1) * 16,
                                                                16)]
                        sums.append(jnp.sum(acc0 + acc1))
                for v in range(BLK * L // 16):
                    vec = jnp.full((16,), sums[v * 16], jnp.float32)
                    for k in range(1, 16):
                        vec = jnp.where(lanes == k, sums[v * 16 + k], vec)
                    ob[pl.ds(bb * (BLK * L) + v * 16, 16)] = vec

            @plsc.parallel_loop(0, CHUNK)
            def retile(r):
                ot[r, pl.ds(0, 16)] = ob[pl.ds(r * L, 16)]
                ot[r, pl.ds(L - 16, 16)] = ob[pl.ds(r * L + L - 16, 16)]

            wb_copy(c, b).start()
        return 0

    lax.fori_loop(0, NCHUNK // 2, pair_body, 0)
    wb_copy(NCHUNK - 2, 0).wait()
    wb_copy(NCHUNK - 1, 1).wait()


def kernel(center, context, in_embed, out_embed):
    center1d = center.astype(jnp.int32)
    ctx2d = context.astype(jnp.int32)

    mesh = plsc.VectorSubcoreMesh(core_axis_name="c", subcore_axis_name="s")
    f = pl.kernel(
        _sc_kernel,
        out_type=jax.ShapeDtypeStruct((B, L), jnp.float32),
        mesh=mesh,
        compiler_params=pltpu.CompilerParams(needs_layout_passes=False),
        scratch_types=[
            pltpu.VMEM((ROWS_PER_W,), jnp.int32),
            pltpu.VMEM((CHUNK, L), jnp.int32),
            pltpu.VMEM((CHUNK, L), jnp.int32),
            pltpu.VMEM((CL,), jnp.int32),
            pltpu.VMEM((CL,), jnp.int32),
            pltpu.VMEM((CHUNK, EMBED), jnp.float32),
            pltpu.VMEM((CHUNK, EMBED), jnp.float32),
            pltpu.VMEM((CL, EMBED), jnp.float32),
            pltpu.VMEM((CL, EMBED), jnp.float32),
            pltpu.VMEM((CL,), jnp.float32),
            pltpu.VMEM((CL,), jnp.float32),
            pltpu.VMEM((CHUNK, L), jnp.float32),
            pltpu.VMEM((CHUNK, L), jnp.float32),
            pltpu.SemaphoreType.DMA,
            pltpu.SemaphoreType.DMA,
            pltpu.SemaphoreType.DMA,
            pltpu.SemaphoreType.DMA,
            pltpu.SemaphoreType.DMA,
            pltpu.SemaphoreType.DMA,
        ],
    )
    return f(center1d, ctx2d, in_embed, out_embed)


# speedup vs baseline: 2.0002x; 1.0035x over previous
import jax
import jax.numpy as jnp
from jax import lax
from jax.experimental import pallas as pl
from jax.experimental.pallas import tpu as pltpu
from jax.experimental.pallas import tpu_sc as plsc

VOCAB = 100000
EMBED = 128
B = 16384
L = 20

NW = 32
ROWS_PER_W = B // NW
CHUNK = 16
CL = CHUNK * L
NCHUNK = ROWS_PER_W // CHUNK
PIECES = (128, 128, 64)
D16 = EMBED // 16
BLK = 4
NBLK = CHUNK // BLK


def _sc_kernel(center1d, ctx2d, in_tab, out_tab, out,
               center_v, cs0, cs1, cf0, cf1, vc0, vc1, vo0, vo1,
               ob0, ob1, ot0, ot1, ssem0, ssem1, gsem0, gsem1,
               wsem0, wsem1):
    wid = lax.axis_index("s") * 2 + lax.axis_index("c")
    lanes = lax.iota(jnp.int32, 16)
    cs_bufs, cf_bufs = (cs0, cs1), (cf0, cf1)
    vc_bufs, vo_bufs = (vc0, vc1), (vo0, vo1)
    ob_bufs, ot_bufs = (ob0, ob1), (ot0, ot1)
    ssems, gsems, wsems = (ssem0, ssem1), (gsem0, gsem1), (wsem0, wsem1)
    row_base = wid * ROWS_PER_W

    pltpu.sync_copy(center1d.at[pl.ds(row_base, ROWS_PER_W)], center_v)

    def stage_copy(ci, b):
        return pltpu.make_async_copy(
            ctx2d.at[pl.ds(row_base + ci * CHUNK, CHUNK)],
            cs_bufs[b], ssems[b])

    def flatten(b):
        cs, cf = cs_bufs[b], cf_bufs[b]
        for r in range(CHUNK):
            a = cs[r, pl.ds(0, 16)]
            z = cs[r, pl.ds(L - 16, 16)]
            cf[pl.ds(r * L, 16)] = a
            cf[pl.ds(r * L + L - 16, 16)] = z

    def gather_copies(ci, b):
        cps = [pltpu.make_async_copy(
            in_tab.at[center_v.at[pl.ds(ci * CHUNK, CHUNK)]],
            vc_bufs[b], gsems[b])]
        off = 0
        for n in PIECES:
            cps.append(pltpu.make_async_copy(
                out_tab.at[cf_bufs[b].at[pl.ds(off, n)]],
                vo_bufs[b].at[pl.ds(off, n)], gsems[b]))
            off += n
        return cps

    def issue_gathers(ci, b):
        for cp in gather_copies(ci, b):
            cp.start()

    def wait_gathers(ci, b):
        for cp in gather_copies(ci, b):
            cp.wait()

    def wb_copy(ci, b):
        return pltpu.make_async_copy(
            ot_bufs[b], out.at[pl.ds(row_base + ci * CHUNK, CHUNK)],
            wsems[b])

    stage_copy(0, 0).start()
    stage_copy(0, 0).wait()
    flatten(0)
    issue_gathers(0, 0)
    stage_copy(1, 1).start()

    def pair_body(c2, _):
        for b in (0, 1):
            c = c2 * 2 + b
            vc_buf, vo_buf = vc_bufs[b], vo_bufs[b]
            ob, ot = ob_bufs[b], ot_bufs[b]

            @pl.when(c + 1 < NCHUNK)
            def _():
                stage_copy(c + 1, 1 - b).wait()
                flatten(1 - b)
                issue_gathers(c + 1, 1 - b)

            @pl.when(c + 2 < NCHUNK)
            def _():
                stage_copy(c + 2, b).start()

            wait_gathers(c, b)

            @pl.when(c >= 2)
            def _():
                wb_copy(c - 2, b).wait()

            @plsc.parallel_loop(0, NBLK)
            def block_body(bb):
                sums = []
                for ii in range(BLK):
                    i = bb * BLK + ii
                    vc = [vc_buf[i, pl.ds(d * 16, 16)] for d in range(D16)]
                    for l in range(L):
                        r = i * L + l
                        acc = vc[0] * vo_buf[r, pl.ds(0, 16)]
                        for d in range(1, D16):
                            acc += vc[d] * vo_buf[r, pl.ds(d * 16, 16)]
                        sums.append(jnp.sum(acc))
                for v in range(BLK * L // 16):
                    vec = jnp.full((16,), sums[v * 16], jnp.float32)
                    for k in range(1, 16):
                        vec = jnp.where(lanes == k, sums[v * 16 + k], vec)
                    ob[pl.ds(bb * (BLK * L) + v * 16, 16)] = vec

            for r in range(CHUNK):
                ot[r, pl.ds(0, 16)] = ob[pl.ds(r * L, 16)]
                ot[r, pl.ds(L - 16, 16)] = ob[pl.ds(r * L + L - 16, 16)]

            wb_copy(c, b).start()
        return 0

    lax.fori_loop(0, NCHUNK // 2, pair_body, 0)
    wb_copy(NCHUNK - 2, 0).wait()
    wb_copy(NCHUNK - 1, 1).wait()


def kernel(center, context, in_embed, out_embed):
    center1d = center.astype(jnp.int32)
    ctx2d = context.astype(jnp.int32)

    mesh = plsc.VectorSubcoreMesh(core_axis_name="c", subcore_axis_name="s")
    f = pl.kernel(
        _sc_kernel,
        out_type=jax.ShapeDtypeStruct((B, L), jnp.float32),
        mesh=mesh,
        compiler_params=pltpu.CompilerParams(needs_layout_passes=False),
        scratch_types=[
            pltpu.VMEM((ROWS_PER_W,), jnp.int32),
            pltpu.VMEM((CHUNK, L), jnp.int32),
            pltpu.VMEM((CHUNK, L), jnp.int32),
            pltpu.VMEM((CL,), jnp.int32),
            pltpu.VMEM((CL,), jnp.int32),
            pltpu.VMEM((CHUNK, EMBED), jnp.float32),
            pltpu.VMEM((CHUNK, EMBED), jnp.float32),
            pltpu.VMEM((CL, EMBED), jnp.float32),
            pltpu.VMEM((CL, EMBED), jnp.float32),
            pltpu.VMEM((CL,), jnp.float32),
            pltpu.VMEM((CL,), jnp.float32),
            pltpu.VMEM((CHUNK, L), jnp.float32),
            pltpu.VMEM((CHUNK, L), jnp.float32),
            pltpu.SemaphoreType.DMA,
            pltpu.SemaphoreType.DMA,
            pltpu.SemaphoreType.DMA,
            pltpu.SemaphoreType.DMA,
            pltpu.SemaphoreType.DMA,
            pltpu.SemaphoreType.DMA,
        ],
    )
    return f(center1d, ctx2d, in_embed, out_embed)
